# baseline (speedup 1.0000x reference)
import functools

import jax
import jax.numpy as jnp
from jax import lax
from jax.experimental import pallas as pl
from jax.experimental.pallas import tpu as pltpu
from jax.experimental.pallas import tpu_sc as plsc

D = 128
EMB = 128
B = 16384
S = 10
NHALF = 2
BH = B // NHALF

NC = 2
NS = 16
NW = NC * NS
CB = 32


def _make_sc_gather(rows_w):
    nchunk = rows_w // CB

    def _sc_body(nodes_hbm, neigh_hbm, feat_hbm, self_hbm, sum_hbm,
                 nidx_v, sidx_v, rows0_v, rows1_v, acc0_v, acc1_v,
                 sbuf0_v, sbuf1_v,
                 semg0, semg1, semso0, semso1, semao0, semao1):
        cid = lax.axis_index("c")
        sid = lax.axis_index("s")
        wid = sid * NC + cid
        base = wid * rows_w
        rows = (rows0_v, rows1_v)
        accs = (acc0_v, acc1_v)
        sbufs = (sbuf0_v, sbuf1_v)
        semg = (semg0, semg1)
        semso = (semso0, semso1)
        semao = (semao0, semao1)

        pltpu.sync_copy(neigh_hbm.at[pl.ds(base * S, rows_w * S)], nidx_v)
        pltpu.sync_copy(nodes_hbm.at[pl.ds(base, rows_w)], sidx_v)

        def start_gathers(c, p):
            descs = []
            for (off, ln) in [(0, 128), (128, 128), (256, 64)]:
                descs.append(pltpu.async_copy(
                    feat_hbm.at[nidx_v.at[pl.ds(c * CB * S + off, ln)]],
                    rows[p].at[pl.ds(off, ln)], semg[p]))
            descs.append(pltpu.async_copy(
                feat_hbm.at[sidx_v.at[pl.ds(c * CB, CB)]], sbufs[p],
                semg[p]))
            return descs

        pending = {}

        def drain(slot):
            d = pending.pop(slot, None)
            if d is not None:
                d.wait()

        inflight = start_gathers(0, 0)
        for c in range(nchunk):
            p = c % 2
            cur = inflight
            if c + 1 < nchunk:
                drain(("so", 1 - p))
                inflight = start_gathers(c + 1, 1 - p)
            for d in cur:
                d.wait()
            pending[("so", p)] = pltpu.async_copy(
                sbufs[p], self_hbm.at[pl.ds(base + c * CB, CB)], semso[p])
            drain(("ao", p))

            rv = rows[p]
            av = accs[p]

            @plsc.parallel_loop(0, CB)
            def _acc(b):
                r0 = b * S
                for g in range(D // 16 // 2):
                    dsls = [pl.ds((2 * g + h) * 16, 16) for h in range(2)]
                    lanes = [[], []]
                    for s in range(S):
                        for h in range(2):
                            lanes[h].append(rv[r0 + s, dsls[h]])
                    while len(lanes[0]) > 1:
                        nxt = [[], []]
                        for i in range(0, len(lanes[0]) - 1, 2):
                            for h in range(2):
                                nxt[h].append(lanes[h][i] + lanes[h][i + 1])
                        if len(lanes[0]) % 2:
                            for h in range(2):
                                nxt[h].append(lanes[h][-1])
                        lanes = nxt
                    for h in range(2):
                        av[b, dsls[h]] = lanes[h][0]

            pending[("ao", p)] = pltpu.async_copy(
                accs[p], sum_hbm.at[pl.ds(base + c * CB, CB)], semao[p])

        for slot in [("so", 0), ("so", 1), ("ao", 0), ("ao", 1)]:
            drain(slot)

    bh = rows_w * NW
    return pl.kernel(
        _sc_body,
        out_type=[jax.ShapeDtypeStruct((bh, D), jnp.float32),
                  jax.ShapeDtypeStruct((bh, D), jnp.float32)],
        mesh=plsc.VectorSubcoreMesh(core_axis_name="c", subcore_axis_name="s"),
        scratch_types=[
            pltpu.VMEM((rows_w * S,), jnp.int32),
            pltpu.VMEM((rows_w,), jnp.int32),
            pltpu.VMEM((CB * S, D), jnp.float32),
            pltpu.VMEM((CB * S, D), jnp.float32),
            pltpu.VMEM((CB, D), jnp.float32),
            pltpu.VMEM((CB, D), jnp.float32),
            pltpu.VMEM((CB, D), jnp.float32),
            pltpu.VMEM((CB, D), jnp.float32),
            pltpu.SemaphoreType.DMA,
            pltpu.SemaphoreType.DMA,
            pltpu.SemaphoreType.DMA,
            pltpu.SemaphoreType.DMA,
            pltpu.SemaphoreType.DMA,
            pltpu.SemaphoreType.DMA,
        ],
    )


_sc_gather_half = _make_sc_gather(BH // NW)

BT = 2048


def _tc_body(w_ref, self_ref, sum_ref, out_ref):
    w1 = w_ref[:, :D]
    w2 = w_ref[:, D:]
    a = lax.dot_general(w1, self_ref[...], (((1,), (1,)), ((), ())),
                        preferred_element_type=jnp.float32)
    b = lax.dot_general(w2, sum_ref[...], (((1,), (1,)), ((), ())),
                        preferred_element_type=jnp.float32)
    out_ref[...] = jnp.maximum(a + b * (1.0 / S), 0.0)


def _tc_body_acc(w_ref, self_ref, sum_ref, prev_ref, out_ref):
    del prev_ref
    _tc_body(w_ref, self_ref, sum_ref, out_ref)


def _tc_matmul_half(weight, self_rows, neigh_sum, half, prev=None):
    col0 = half * (BH // BT)
    in_specs = [
        pl.BlockSpec((EMB, 2 * D), lambda i: (0, 0)),
        pl.BlockSpec((BT, D), lambda i: (i, 0)),
        pl.BlockSpec((BT, D), lambda i: (i, 0)),
    ]
    args = [weight, self_rows, neigh_sum]
    body = _tc_body
    aliases = {}
    if prev is not None:
        in_specs.append(pl.BlockSpec(memory_space=pl.ANY))
        args.append(prev)
        body = _tc_body_acc
        aliases = {3: 0}
    return pl.pallas_call(
        body,
        grid=(BH // BT,),
        in_specs=in_specs,
        out_specs=pl.BlockSpec((EMB, BT), lambda i: (0, i + col0)),
        out_shape=jax.ShapeDtypeStruct((EMB, B), jnp.float32),
        input_output_aliases=aliases,
    )(*args)


def kernel(nodes, neigh_idx, features, weight):
    halves = []
    for h in range(NHALF):
        lo, hi = h * BH, (h + 1) * BH
        halves.append(_sc_gather_half(
            nodes[lo:hi], neigh_idx[lo:hi].reshape(-1), features))
    out = None
    for h, (self_rows, neigh_sum) in enumerate(halves):
        out = _tc_matmul_half(weight, self_rows, neigh_sum, h, prev=out)
    return out


# speedup vs baseline: 1.0084x; 1.0084x over previous
import functools

import jax
import jax.numpy as jnp
from jax import lax
from jax.experimental import pallas as pl
from jax.experimental.pallas import tpu as pltpu
from jax.experimental.pallas import tpu_sc as plsc

D = 128
EMB = 128
B = 16384
S = 10
NHALF = 1
BH = B // NHALF

NC = 2
NS = 16
NW = NC * NS
CB = 32


def _make_sc_gather(rows_w):
    nchunk = rows_w // CB

    def _sc_body(nodes_hbm, neigh_hbm, feat_hbm, self_hbm, sum_hbm,
                 nidx_v, sidx_v, rows0_v, rows1_v, acc0_v, acc1_v,
                 sbuf0_v, sbuf1_v,
                 semg0, semg1, semso0, semso1, semao0, semao1):
        cid = lax.axis_index("c")
        sid = lax.axis_index("s")
        wid = sid * NC + cid
        base = wid * rows_w
        rows = (rows0_v, rows1_v)
        accs = (acc0_v, acc1_v)
        sbufs = (sbuf0_v, sbuf1_v)
        semg = (semg0, semg1)
        semso = (semso0, semso1)
        semao = (semao0, semao1)

        pltpu.sync_copy(neigh_hbm.at[pl.ds(base * S, rows_w * S)], nidx_v)
        pltpu.sync_copy(nodes_hbm.at[pl.ds(base, rows_w)], sidx_v)

        def start_gathers(c, p):
            descs = []
            for (off, ln) in [(0, 128), (128, 128), (256, 64)]:
                descs.append(pltpu.async_copy(
                    feat_hbm.at[nidx_v.at[pl.ds(c * CB * S + off, ln)]],
                    rows[p].at[pl.ds(off, ln)], semg[p]))
            descs.append(pltpu.async_copy(
                feat_hbm.at[sidx_v.at[pl.ds(c * CB, CB)]], sbufs[p],
                semg[p]))
            return descs

        pending = {}

        def drain(slot):
            d = pending.pop(slot, None)
            if d is not None:
                d.wait()

        inflight = start_gathers(0, 0)
        for c in range(nchunk):
            p = c % 2
            cur = inflight
            if c + 1 < nchunk:
                drain(("so", 1 - p))
                inflight = start_gathers(c + 1, 1 - p)
            for d in cur:
                d.wait()
            pending[("so", p)] = pltpu.async_copy(
                sbufs[p], self_hbm.at[pl.ds(base + c * CB, CB)], semso[p])
            drain(("ao", p))

            rv = rows[p]
            av = accs[p]

            @plsc.parallel_loop(0, CB)
            def _acc(b):
                r0 = b * S
                for g in range(D // 16 // 2):
                    dsls = [pl.ds((2 * g + h) * 16, 16) for h in range(2)]
                    lanes = [[], []]
                    for s in range(S):
                        for h in range(2):
                            lanes[h].append(rv[r0 + s, dsls[h]])
                    while len(lanes[0]) > 1:
                        nxt = [[], []]
                        for i in range(0, len(lanes[0]) - 1, 2):
                            for h in range(2):
                                nxt[h].append(lanes[h][i] + lanes[h][i + 1])
                        if len(lanes[0]) % 2:
                            for h in range(2):
                                nxt[h].append(lanes[h][-1])
                        lanes = nxt
                    for h in range(2):
                        av[b, dsls[h]] = lanes[h][0]

            pending[("ao", p)] = pltpu.async_copy(
                accs[p], sum_hbm.at[pl.ds(base + c * CB, CB)], semao[p])

        for slot in [("so", 0), ("so", 1), ("ao", 0), ("ao", 1)]:
            drain(slot)

    bh = rows_w * NW
    return pl.kernel(
        _sc_body,
        out_type=[jax.ShapeDtypeStruct((bh, D), jnp.float32),
                  jax.ShapeDtypeStruct((bh, D), jnp.float32)],
        mesh=plsc.VectorSubcoreMesh(core_axis_name="c", subcore_axis_name="s"),
        scratch_types=[
            pltpu.VMEM((rows_w * S,), jnp.int32),
            pltpu.VMEM((rows_w,), jnp.int32),
            pltpu.VMEM((CB * S, D), jnp.float32),
            pltpu.VMEM((CB * S, D), jnp.float32),
            pltpu.VMEM((CB, D), jnp.float32),
            pltpu.VMEM((CB, D), jnp.float32),
            pltpu.VMEM((CB, D), jnp.float32),
            pltpu.VMEM((CB, D), jnp.float32),
            pltpu.SemaphoreType.DMA,
            pltpu.SemaphoreType.DMA,
            pltpu.SemaphoreType.DMA,
            pltpu.SemaphoreType.DMA,
            pltpu.SemaphoreType.DMA,
            pltpu.SemaphoreType.DMA,
        ],
    )


_sc_gather_half = _make_sc_gather(BH // NW)

BT = 2048


def _tc_body(w_ref, self_ref, sum_ref, out_ref):
    w1 = w_ref[:, :D]
    w2 = w_ref[:, D:]
    a = lax.dot_general(w1, self_ref[...], (((1,), (1,)), ((), ())),
                        preferred_element_type=jnp.float32)
    b = lax.dot_general(w2, sum_ref[...], (((1,), (1,)), ((), ())),
                        preferred_element_type=jnp.float32)
    out_ref[...] = jnp.maximum(a + b * (1.0 / S), 0.0)


def _tc_body_acc(w_ref, self_ref, sum_ref, prev_ref, out_ref):
    del prev_ref
    _tc_body(w_ref, self_ref, sum_ref, out_ref)


def _tc_matmul_half(weight, self_rows, neigh_sum, half, prev=None):
    col0 = half * (BH // BT)
    in_specs = [
        pl.BlockSpec((EMB, 2 * D), lambda i: (0, 0)),
        pl.BlockSpec((BT, D), lambda i: (i, 0)),
        pl.BlockSpec((BT, D), lambda i: (i, 0)),
    ]
    args = [weight, self_rows, neigh_sum]
    body = _tc_body
    aliases = {}
    if prev is not None:
        in_specs.append(pl.BlockSpec(memory_space=pl.ANY))
        args.append(prev)
        body = _tc_body_acc
        aliases = {3: 0}
    return pl.pallas_call(
        body,
        grid=(BH // BT,),
        in_specs=in_specs,
        out_specs=pl.BlockSpec((EMB, BT), lambda i: (0, i + col0)),
        out_shape=jax.ShapeDtypeStruct((EMB, B), jnp.float32),
        input_output_aliases=aliases,
    )(*args)


def kernel(nodes, neigh_idx, features, weight):
    halves = []
    for h in range(NHALF):
        lo, hi = h * BH, (h + 1) * BH
        halves.append(_sc_gather_half(
            nodes[lo:hi], neigh_idx[lo:hi].reshape(-1), features))
    out = None
    for h, (self_rows, neigh_sum) in enumerate(halves):
        out = _tc_matmul_half(weight, self_rows, neigh_sum, h, prev=out)
    return out
